# baseline (speedup 1.0000x reference)
import functools

import jax
import jax.numpy as jnp
from jax import lax
from jax.experimental import pallas as pl
from jax.experimental.pallas import tpu as pltpu
from jax.experimental.pallas import tpu_sc as plsc

N_VERT = 6890
N_PAD = 6912
Q = 16384
B = 512
K = 4
LAT = 64
HID = 128
TW = 128
BIG = 3.0e38


BQ = 512
NCH = N_PAD // 128
_IBIG = 0x7F000000


def _topk_body(qb_ref, vt_ref, idx_ref):
    qb = qb_ref[...]
    v = vt_ref[...]
    qsq = (qb[:, 0:1] * qb[:, 0:1] + qb[:, 1:2] * qb[:, 1:2]
           + qb[:, 2:3] * qb[:, 2:3])
    qbn = -2.0 * qb
    q0 = qbn[:, 0:1]
    q1 = qbn[:, 1:2]
    q2 = qbn[:, 2:3]
    vsq = (v[0:1, :] * v[0:1, :] + v[1:2, :] * v[1:2, :]
           + v[2:3, :] * v[2:3, :])
    big = jnp.full((BQ, 128), BIG, jnp.float32)
    y1 = big
    y2 = big
    y3 = big
    y4 = big
    for c in range(NCH):
        sl = slice(c * 128, (c + 1) * 128)
        x = (vsq[:, sl] + qsq) + q0 * v[0:1, sl]
        x = x + q1 * v[1:2, sl]
        x = x + q2 * v[2:3, sl]
        y = lax.bitcast_convert_type(
            (lax.bitcast_convert_type(x, jnp.int32) & jnp.int32(0x7FFFFFC0))
            | jnp.int32(c), jnp.float32)
        y1n = jnp.minimum(y1, y)
        r = jnp.maximum(y1, y)
        y2n = jnp.minimum(y2, r)
        r2 = jnp.maximum(y2, r)
        y3n = jnp.minimum(y3, r2)
        r3 = jnp.maximum(y3, r2)
        y4 = jnp.minimum(y4, r3)
        y1, y2, y3 = y1n, y2n, y3n
    tw = jnp.concatenate([y1, y2, y3, y4], axis=1)
    lane = lax.broadcasted_iota(jnp.int32, (BQ, 512), 1) & 127
    gidx = (lax.bitcast_convert_type(tw, jnp.int32) & 63) * 128 + lane
    idxs = []
    for _ in range(K):
        m = jnp.min(tw, axis=1, keepdims=True)
        sel = tw == m
        ik = jnp.min(jnp.where(sel, gidx, N_PAD), axis=1, keepdims=True)
        idxs.append(ik)
        chosen = sel & (gidx == ik)
        tw = jnp.where(chosen, BIG, tw)
    idx_ref[...] = jnp.concatenate(
        [jnp.reshape(ik, (1, BQ)) for ik in idxs], axis=0)


def _topk(query_pts, vt, nq):
    grid = nq // BQ
    return pl.pallas_call(
        _topk_body,
        grid=(grid,),
        in_specs=[
            pl.BlockSpec((BQ, 3), lambda i: (i, 0)),
            pl.BlockSpec((3, N_PAD), lambda i: (0, 0)),
        ],
        out_specs=pl.BlockSpec((K, BQ), lambda i: (0, i)),
        out_shape=jax.ShapeDtypeStruct((K, nq), jnp.int32),
    )(query_pts, vt)


_NW = 32
_CHUNK = 512


def _sc_gather(idx_flat, tab, nrows):
    rows_per_w = nrows // _NW
    mesh = plsc.VectorSubcoreMesh(core_axis_name="c", subcore_axis_name="s")

    @functools.partial(
        pl.kernel, mesh=mesh,
        out_type=jax.ShapeDtypeStruct((nrows, TW), jnp.float32),
        scratch_types=[
            pltpu.VMEM((_CHUNK,), jnp.int32),
            pltpu.VMEM((_CHUNK, TW), jnp.float32),
            pltpu.SemaphoreType.DMA,
        ],
    )
    def gather_kernel(idx_hbm, tab_hbm, out_hbm, idx_v, rows_v, sem):
        wid = lax.axis_index("s") * 2 + lax.axis_index("c")
        for c in range(rows_per_w // _CHUNK):
            base = wid * rows_per_w + c * _CHUNK
            pltpu.sync_copy(idx_hbm.at[pl.ds(base, _CHUNK)], idx_v)
            pltpu.async_copy(tab_hbm.at[idx_v], rows_v, sem).wait()
            pltpu.sync_copy(rows_v, out_hbm.at[pl.ds(base, _CHUNK)])

    return gather_kernel(idx_flat, tab)


def _mlp_body(qb_ref, g0_ref, g1_ref, g2_ref, g3_ref, w1f_ref, w1v_ref,
              b1_ref, w2_ref, b2_ref, w3_ref, b3_ref, w4_ref, b4_ref, out_ref):
    qb = qb_ref[...]
    acc = jnp.zeros((B, LAT), jnp.float32)
    invsum = jnp.zeros((B, 1), jnp.float32)
    for g_ref in (g0_ref, g1_ref, g2_ref, g3_ref):
        g = g_ref[...]
        fk = g[:, 0:LAT]
        vk = g[:, LAT:LAT + 3]
        xv = qb - vk
        d2 = (xv[:, 0:1] * xv[:, 0:1] + xv[:, 1:2] * xv[:, 1:2]
              + xv[:, 2:3] * xv[:, 2:3])
        dist = jnp.sqrt(jnp.maximum(d2, 1e-12))
        inv = 1.0 / (dist + 1e-9)
        pre = jnp.dot(fk, w1f_ref[...], preferred_element_type=jnp.float32)
        pre = (pre
               + xv[:, 0:1] * w1v_ref[0:1, :]
               + xv[:, 1:2] * w1v_ref[1:2, :]
               + xv[:, 2:3] * w1v_ref[2:3, :]
               + b1_ref[...])
        h = jnp.maximum(pre, 0.0)
        h = jnp.maximum(jnp.dot(h, w2_ref[...], preferred_element_type=jnp.float32)
                        + b2_ref[...], 0.0)
        h = jnp.maximum(jnp.dot(h, w3_ref[...], preferred_element_type=jnp.float32)
                        + b3_ref[...], 0.0)
        f = jnp.dot(h, w4_ref[...], preferred_element_type=jnp.float32) + b4_ref[...]
        invsum = invsum + inv
        acc = acc + inv * f
    out_ref[...] = acc / invsum


def _mlp(query_pts, gathered, w1f, w1v, b1, W2, b2, W3, b3, W4, b4, nq):
    grid = nq // B
    full = lambda shape: pl.BlockSpec(shape, lambda i: (0,) * len(shape))

    def gspec(k):
        return pl.BlockSpec((B, TW), lambda i, k=k: (k * grid + i, 0))

    return pl.pallas_call(
        _mlp_body,
        grid=(grid,),
        in_specs=[
            pl.BlockSpec((B, 3), lambda i: (i, 0)),
            gspec(0), gspec(1), gspec(2), gspec(3),
            full((LAT, HID)), full((3, HID)), full((1, HID)),
            full((HID, HID)), full((1, HID)),
            full((HID, HID)), full((1, HID)),
            full((HID, LAT)), full((1, LAT)),
        ],
        out_specs=pl.BlockSpec((B, LAT), lambda i: (i, 0)),
        out_shape=jax.ShapeDtypeStruct((nq, LAT), jnp.float32),
    )(query_pts, gathered, gathered, gathered, gathered,
      w1f, w1v, b1.reshape(1, HID), W2, b2.reshape(1, HID),
      W3, b3.reshape(1, HID), W4, b4.reshape(1, LAT))


def kernel(vertex, vertex_features, query_pts, W1, b1, W2, b2, W3, b3, W4, b4):
    pad = N_PAD - N_VERT
    vert_far = jnp.pad(vertex, ((0, pad), (0, 0)), constant_values=1.0e18)
    vt = vert_far.T
    tab = jnp.concatenate(
        [jnp.pad(vertex_features, ((0, pad), (0, 0))),
         jnp.pad(vertex, ((0, pad), (0, 0))),
         jnp.zeros((N_PAD, TW - LAT - 3), jnp.float32)], axis=1)

    nh = Q // 4
    outs = []
    for h in range(4):
        qh = lax.slice_in_dim(query_pts, h * nh, (h + 1) * nh, axis=0)
        idx = _topk(qh, vt, nh)
        gathered = _sc_gather(idx.reshape(nh * K), tab, nh * K)
        outs.append(_mlp(qh, gathered, W1[:LAT, :], W1[LAT:, :],
                         b1, W2, b2, W3, b3, W4, b4, nh))
    return jnp.concatenate(outs, axis=0)


# speedup vs baseline: 1.0292x; 1.0292x over previous
import functools

import jax
import jax.numpy as jnp
from jax import lax
from jax.experimental import pallas as pl
from jax.experimental.pallas import tpu as pltpu
from jax.experimental.pallas import tpu_sc as plsc

N_VERT = 6890
N_PAD = 6912
Q = 16384
B = 512
K = 4
LAT = 64
HID = 128
TW = 128
BIG = 3.0e38


BQ = 1024
NCH = N_PAD // 128
_IBIG = 0x7F000000


def _topk_body(qb_ref, vt_ref, idx_ref):
    qb = qb_ref[...]
    v = vt_ref[...]
    qsq = (qb[:, 0:1] * qb[:, 0:1] + qb[:, 1:2] * qb[:, 1:2]
           + qb[:, 2:3] * qb[:, 2:3])
    qbn = -2.0 * qb
    q0 = qbn[:, 0:1]
    q1 = qbn[:, 1:2]
    q2 = qbn[:, 2:3]
    vsq = (v[0:1, :] * v[0:1, :] + v[1:2, :] * v[1:2, :]
           + v[2:3, :] * v[2:3, :])
    big = jnp.full((BQ, 128), BIG, jnp.float32)
    y1 = big
    y2 = big
    y3 = big
    y4 = big
    for c in range(NCH):
        sl = slice(c * 128, (c + 1) * 128)
        x = (vsq[:, sl] + qsq) + q0 * v[0:1, sl]
        x = x + q1 * v[1:2, sl]
        x = x + q2 * v[2:3, sl]
        y = lax.bitcast_convert_type(
            (lax.bitcast_convert_type(x, jnp.int32) & jnp.int32(0x7FFFFFC0))
            | jnp.int32(c), jnp.float32)
        y1n = jnp.minimum(y1, y)
        r = jnp.maximum(y1, y)
        y2n = jnp.minimum(y2, r)
        r2 = jnp.maximum(y2, r)
        y3n = jnp.minimum(y3, r2)
        r3 = jnp.maximum(y3, r2)
        y4 = jnp.minimum(y4, r3)
        y1, y2, y3 = y1n, y2n, y3n
    tw = jnp.concatenate([y1, y2, y3, y4], axis=1)
    lane = lax.broadcasted_iota(jnp.int32, (BQ, 512), 1) & 127
    gidx = (lax.bitcast_convert_type(tw, jnp.int32) & 63) * 128 + lane
    idxs = []
    for _ in range(K):
        m = jnp.min(tw, axis=1, keepdims=True)
        sel = tw == m
        ik = jnp.min(jnp.where(sel, gidx, N_PAD), axis=1, keepdims=True)
        idxs.append(ik)
        chosen = sel & (gidx == ik)
        tw = jnp.where(chosen, BIG, tw)
    idx_ref[...] = jnp.concatenate(
        [jnp.reshape(ik, (1, BQ)) for ik in idxs], axis=0)


def _topk(query_pts, vt, nq):
    grid = nq // BQ
    return pl.pallas_call(
        _topk_body,
        grid=(grid,),
        in_specs=[
            pl.BlockSpec((BQ, 3), lambda i: (i, 0)),
            pl.BlockSpec((3, N_PAD), lambda i: (0, 0)),
        ],
        out_specs=pl.BlockSpec((K, BQ), lambda i: (0, i)),
        out_shape=jax.ShapeDtypeStruct((K, nq), jnp.int32),
    )(query_pts, vt)


_NW = 32
_CHUNK = 512


def _sc_gather(idx_flat, tab, nrows):
    rows_per_w = nrows // _NW
    mesh = plsc.VectorSubcoreMesh(core_axis_name="c", subcore_axis_name="s")

    @functools.partial(
        pl.kernel, mesh=mesh,
        out_type=jax.ShapeDtypeStruct((nrows, TW), jnp.float32),
        scratch_types=[
            pltpu.VMEM((_CHUNK,), jnp.int32),
            pltpu.VMEM((_CHUNK, TW), jnp.float32),
            pltpu.SemaphoreType.DMA,
        ],
    )
    def gather_kernel(idx_hbm, tab_hbm, out_hbm, idx_v, rows_v, sem):
        wid = lax.axis_index("s") * 2 + lax.axis_index("c")
        for c in range(rows_per_w // _CHUNK):
            base = wid * rows_per_w + c * _CHUNK
            pltpu.sync_copy(idx_hbm.at[pl.ds(base, _CHUNK)], idx_v)
            pltpu.async_copy(tab_hbm.at[idx_v], rows_v, sem).wait()
            pltpu.sync_copy(rows_v, out_hbm.at[pl.ds(base, _CHUNK)])

    return gather_kernel(idx_flat, tab)


def _mlp_body(qb_ref, g0_ref, g1_ref, g2_ref, g3_ref, w1f_ref, w1v_ref,
              b1_ref, w2_ref, b2_ref, w3_ref, b3_ref, w4_ref, b4_ref, out_ref):
    qb = qb_ref[...]
    acc = jnp.zeros((B, LAT), jnp.float32)
    invsum = jnp.zeros((B, 1), jnp.float32)
    for g_ref in (g0_ref, g1_ref, g2_ref, g3_ref):
        g = g_ref[...]
        fk = g[:, 0:LAT]
        vk = g[:, LAT:LAT + 3]
        xv = qb - vk
        d2 = (xv[:, 0:1] * xv[:, 0:1] + xv[:, 1:2] * xv[:, 1:2]
              + xv[:, 2:3] * xv[:, 2:3])
        dist = jnp.sqrt(jnp.maximum(d2, 1e-12))
        inv = 1.0 / (dist + 1e-9)
        pre = jnp.dot(fk, w1f_ref[...], preferred_element_type=jnp.float32)
        pre = (pre
               + xv[:, 0:1] * w1v_ref[0:1, :]
               + xv[:, 1:2] * w1v_ref[1:2, :]
               + xv[:, 2:3] * w1v_ref[2:3, :]
               + b1_ref[...])
        h = jnp.maximum(pre, 0.0)
        h = jnp.maximum(jnp.dot(h, w2_ref[...], preferred_element_type=jnp.float32)
                        + b2_ref[...], 0.0)
        h = jnp.maximum(jnp.dot(h, w3_ref[...], preferred_element_type=jnp.float32)
                        + b3_ref[...], 0.0)
        f = jnp.dot(h, w4_ref[...], preferred_element_type=jnp.float32) + b4_ref[...]
        invsum = invsum + inv
        acc = acc + inv * f
    out_ref[...] = acc / invsum


def _mlp(query_pts, gathered, w1f, w1v, b1, W2, b2, W3, b3, W4, b4, nq):
    grid = nq // B
    full = lambda shape: pl.BlockSpec(shape, lambda i: (0,) * len(shape))

    def gspec(k):
        return pl.BlockSpec((B, TW), lambda i, k=k: (k * grid + i, 0))

    return pl.pallas_call(
        _mlp_body,
        grid=(grid,),
        in_specs=[
            pl.BlockSpec((B, 3), lambda i: (i, 0)),
            gspec(0), gspec(1), gspec(2), gspec(3),
            full((LAT, HID)), full((3, HID)), full((1, HID)),
            full((HID, HID)), full((1, HID)),
            full((HID, HID)), full((1, HID)),
            full((HID, LAT)), full((1, LAT)),
        ],
        out_specs=pl.BlockSpec((B, LAT), lambda i: (i, 0)),
        out_shape=jax.ShapeDtypeStruct((nq, LAT), jnp.float32),
    )(query_pts, gathered, gathered, gathered, gathered,
      w1f, w1v, b1.reshape(1, HID), W2, b2.reshape(1, HID),
      W3, b3.reshape(1, HID), W4, b4.reshape(1, LAT))


def kernel(vertex, vertex_features, query_pts, W1, b1, W2, b2, W3, b3, W4, b4):
    pad = N_PAD - N_VERT
    vert_far = jnp.pad(vertex, ((0, pad), (0, 0)), constant_values=1.0e18)
    vt = vert_far.T
    tab = jnp.concatenate(
        [jnp.pad(vertex_features, ((0, pad), (0, 0))),
         jnp.pad(vertex, ((0, pad), (0, 0))),
         jnp.zeros((N_PAD, TW - LAT - 3), jnp.float32)], axis=1)

    nh = Q // 2
    outs = []
    for h in range(2):
        qh = lax.slice_in_dim(query_pts, h * nh, (h + 1) * nh, axis=0)
        idx = _topk(qh, vt, nh)
        gathered = _sc_gather(idx.reshape(nh * K), tab, nh * K)
        outs.append(_mlp(qh, gathered, W1[:LAT, :], W1[LAT:, :],
                         b1, W2, b2, W3, b3, W4, b4, nh))
    return jnp.concatenate(outs, axis=0)


# speedup vs baseline: 1.0521x; 1.0222x over previous
import functools

import jax
import jax.numpy as jnp
from jax import lax
from jax.experimental import pallas as pl
from jax.experimental.pallas import tpu as pltpu
from jax.experimental.pallas import tpu_sc as plsc

N_VERT = 6890
N_PAD = 6912
Q = 16384
B = 512
K = 4
LAT = 64
HID = 128
TW = 128
BIG = 3.0e38


BQ = 512
NCH = N_PAD // 128
_IBIG = 0x7F000000


def _topk_body(qb_ref, vt_ref, idx_ref):
    qb = qb_ref[...]
    v = vt_ref[...]
    qsq = (qb[:, 0:1] * qb[:, 0:1] + qb[:, 1:2] * qb[:, 1:2]
           + qb[:, 2:3] * qb[:, 2:3])
    qbn = -2.0 * qb
    q0 = qbn[:, 0:1]
    q1 = qbn[:, 1:2]
    q2 = qbn[:, 2:3]
    vsq = (v[0:1, :] * v[0:1, :] + v[1:2, :] * v[1:2, :]
           + v[2:3, :] * v[2:3, :])
    big = jnp.full((BQ, 128), BIG, jnp.float32)
    y1 = big
    y2 = big
    y3 = big
    y4 = big
    for c in range(NCH):
        sl = slice(c * 128, (c + 1) * 128)
        x = (vsq[:, sl] + qsq) + q0 * v[0:1, sl]
        x = x + q1 * v[1:2, sl]
        x = x + q2 * v[2:3, sl]
        y = lax.bitcast_convert_type(
            (lax.bitcast_convert_type(x, jnp.int32) & jnp.int32(0x7FFFFFC0))
            | jnp.int32(c), jnp.float32)
        y1n = jnp.minimum(y1, y)
        r = jnp.maximum(y1, y)
        y2n = jnp.minimum(y2, r)
        r2 = jnp.maximum(y2, r)
        y3n = jnp.minimum(y3, r2)
        r3 = jnp.maximum(y3, r2)
        y4 = jnp.minimum(y4, r3)
        y1, y2, y3 = y1n, y2n, y3n
    tw = jnp.concatenate([y1, y2, y3, y4], axis=1)
    lane = lax.broadcasted_iota(jnp.int32, (BQ, 512), 1) & 127
    gidx = (lax.bitcast_convert_type(tw, jnp.int32) & 63) * 128 + lane
    idxs = []
    for _ in range(K):
        m = jnp.min(tw, axis=1, keepdims=True)
        sel = tw == m
        ik = jnp.min(jnp.where(sel, gidx, N_PAD), axis=1, keepdims=True)
        idxs.append(ik)
        chosen = sel & (gidx == ik)
        tw = jnp.where(chosen, BIG, tw)
    idx_ref[...] = jnp.concatenate(
        [jnp.reshape(ik, (1, BQ)) for ik in idxs], axis=0)


def _topk(query_pts, vt, nq):
    grid = nq // BQ
    return pl.pallas_call(
        _topk_body,
        grid=(grid,),
        in_specs=[
            pl.BlockSpec((BQ, 3), lambda i: (i, 0)),
            pl.BlockSpec((3, N_PAD), lambda i: (0, 0)),
        ],
        out_specs=pl.BlockSpec((K, BQ), lambda i: (0, i)),
        out_shape=jax.ShapeDtypeStruct((K, nq), jnp.int32),
    )(query_pts, vt)


_NW = 32
_CHUNK = 512


def _sc_gather(idx_flat, tab, nrows):
    rows_per_w = nrows // _NW
    mesh = plsc.VectorSubcoreMesh(core_axis_name="c", subcore_axis_name="s")

    @functools.partial(
        pl.kernel, mesh=mesh,
        out_type=jax.ShapeDtypeStruct((nrows, TW), jnp.float32),
        scratch_types=[
            pltpu.VMEM((_CHUNK,), jnp.int32),
            pltpu.VMEM((_CHUNK, TW), jnp.float32),
            pltpu.SemaphoreType.DMA,
        ],
    )
    def gather_kernel(idx_hbm, tab_hbm, out_hbm, idx_v, rows_v, sem):
        wid = lax.axis_index("s") * 2 + lax.axis_index("c")
        for c in range(rows_per_w // _CHUNK):
            base = wid * rows_per_w + c * _CHUNK
            pltpu.sync_copy(idx_hbm.at[pl.ds(base, _CHUNK)], idx_v)
            pltpu.async_copy(tab_hbm.at[idx_v], rows_v, sem).wait()
            pltpu.sync_copy(rows_v, out_hbm.at[pl.ds(base, _CHUNK)])

    return gather_kernel(idx_flat, tab)


def _mlp_body(qb_ref, g0_ref, g1_ref, g2_ref, g3_ref, w1f_ref, w1v_ref,
              b1_ref, w2_ref, b2_ref, w3_ref, b3_ref, w4_ref, b4_ref, out_ref):
    qb = qb_ref[...]
    acc = jnp.zeros((B, LAT), jnp.float32)
    invsum = jnp.zeros((B, 1), jnp.float32)
    for g_ref in (g0_ref, g1_ref, g2_ref, g3_ref):
        g = g_ref[...]
        fk = g[:, 0:LAT]
        vk = g[:, LAT:LAT + 3]
        xv = qb - vk
        d2 = (xv[:, 0:1] * xv[:, 0:1] + xv[:, 1:2] * xv[:, 1:2]
              + xv[:, 2:3] * xv[:, 2:3])
        dist = jnp.sqrt(jnp.maximum(d2, 1e-12))
        inv = 1.0 / (dist + 1e-9)
        pre = jnp.dot(fk, w1f_ref[...], preferred_element_type=jnp.float32)
        pre = (pre
               + xv[:, 0:1] * w1v_ref[0:1, :]
               + xv[:, 1:2] * w1v_ref[1:2, :]
               + xv[:, 2:3] * w1v_ref[2:3, :]
               + b1_ref[...])
        h = jnp.maximum(pre, 0.0)
        h = jnp.maximum(jnp.dot(h, w2_ref[...], preferred_element_type=jnp.float32)
                        + b2_ref[...], 0.0)
        h = jnp.maximum(jnp.dot(h, w3_ref[...], preferred_element_type=jnp.float32)
                        + b3_ref[...], 0.0)
        f = jnp.dot(h, w4_ref[...], preferred_element_type=jnp.float32) + b4_ref[...]
        invsum = invsum + inv
        acc = acc + inv * f
    out_ref[...] = acc / invsum


def _mlp(query_pts, gathered, w1f, w1v, b1, W2, b2, W3, b3, W4, b4, nq):
    grid = nq // B
    full = lambda shape: pl.BlockSpec(shape, lambda i: (0,) * len(shape))

    def gspec(k):
        return pl.BlockSpec((B, TW), lambda i, k=k: (k * grid + i, 0))

    return pl.pallas_call(
        _mlp_body,
        grid=(grid,),
        in_specs=[
            pl.BlockSpec((B, 3), lambda i: (i, 0)),
            gspec(0), gspec(1), gspec(2), gspec(3),
            full((LAT, HID)), full((3, HID)), full((1, HID)),
            full((HID, HID)), full((1, HID)),
            full((HID, HID)), full((1, HID)),
            full((HID, LAT)), full((1, LAT)),
        ],
        out_specs=pl.BlockSpec((B, LAT), lambda i: (i, 0)),
        out_shape=jax.ShapeDtypeStruct((nq, LAT), jnp.float32),
    )(query_pts, gathered, gathered, gathered, gathered,
      w1f, w1v, b1.reshape(1, HID), W2, b2.reshape(1, HID),
      W3, b3.reshape(1, HID), W4, b4.reshape(1, LAT))


def kernel(vertex, vertex_features, query_pts, W1, b1, W2, b2, W3, b3, W4, b4):
    pad = N_PAD - N_VERT
    vert_far = jnp.pad(vertex, ((0, pad), (0, 0)), constant_values=1.0e18)
    vt = vert_far.T
    tab = jnp.concatenate(
        [jnp.pad(vertex_features, ((0, pad), (0, 0))),
         jnp.pad(vertex, ((0, pad), (0, 0))),
         jnp.zeros((N_PAD, TW - LAT - 3), jnp.float32)], axis=1)

    nh = Q // 2
    outs = []
    for h in range(2):
        qh = lax.slice_in_dim(query_pts, h * nh, (h + 1) * nh, axis=0)
        idx = _topk(qh, vt, nh)
        gathered = _sc_gather(idx.reshape(nh * K), tab, nh * K)
        outs.append(_mlp(qh, gathered, W1[:LAT, :], W1[LAT:, :],
                         b1, W2, b2, W3, b3, W4, b4, nh))
    return jnp.concatenate(outs, axis=0)
